# baseline (speedup 1.0000x reference)
import jax
import jax.numpy as jnp
from jax import lax
from jax.experimental import pallas as pl
from jax.experimental.pallas import tpu as pltpu
from jax.experimental.pallas import tpu_sc as plsc

F = 26
D = 64
P = F * (F - 1) // 2
L = 16
NC = 2
NS = 16
NW = NC * NS
XW = L * F * D
OW = L * P
H = XW // 2

_PAIR_IDX = {}
for _i in range(F - 1):
    for _j in range(_i + 1, F):
        _PAIR_IDX[(_i, _j)] = len(_PAIR_IDX)

_BLOCKS = [(0, 6), (6, 12), (12, 18), (18, 24), (24, 26)]

_SCHED = []
for _bi in range(len(_BLOCKS)):
    _fi = list(range(*_BLOCKS[_bi]))
    _diag = [(i, j) for i in _fi for j in _fi if i < j]
    if _diag:
        _SCHED.append((_fi, _diag))
    for _bj in range(_bi + 1, len(_BLOCKS)):
        _fj = list(range(*_BLOCKS[_bj]))
        _SCHED.append((_fi + _fj, [(i, j) for i in _fi for j in _fj]))

assert sorted(p for _, ps in _SCHED for p in ps) == sorted(_PAIR_IDX)


def _body(b_total):
    rows_per_w = b_total // NW
    nchunks = rows_per_w // L
    nphase = nchunks // 2

    def body(x_hbm, out_hbm, x_v0, x_v1, out_v0, out_v1,
             in_sa0, in_sb0, in_sa1, in_sb1, out_s0, out_s1):
        wid = lax.axis_index("s") * NC + lax.axis_index("c")
        base = wid * rows_per_w
        b_iota = lax.iota(jnp.int32, L)
        bx = b_iota * (F * D)
        bo = b_iota * P

        def issue_in(cidx, x_v, sa, sb):
            off = (base + cidx * L) * (F * D)
            pltpu.async_copy(x_hbm.at[pl.ds(off, H)], x_v.at[pl.ds(0, H)], sa)
            pltpu.async_copy(x_hbm.at[pl.ds(off + H, H)],
                             x_v.at[pl.ds(H, H)], sb)

        def wait_in(x_v, sa, sb):
            pltpu.make_async_copy(x_hbm.at[pl.ds(0, H)],
                                  x_v.at[pl.ds(0, H)], sa).wait()
            pltpu.make_async_copy(x_hbm.at[pl.ds(0, H)],
                                  x_v.at[pl.ds(H, H)], sb).wait()

        def compute(x_v, out_v):
            for fields, pairs in _SCHED:
                fbase = {f: bx + f * D for f in fields}

                def dstep(d, accs, fields=fields, pairs=pairs, fbase=fbase):
                    rot = (jnp.full((L,), d, jnp.int32) + b_iota) & (D - 1)
                    vals = {
                        f: plsc.load_gather(x_v, [fbase[f] + rot])
                        for f in fields
                    }
                    return tuple(a + vals[i] * vals[j]
                                 for a, (i, j) in zip(accs, pairs))

                accs = lax.fori_loop(
                    0, D, dstep,
                    tuple(jnp.zeros((L,), jnp.float32) for _ in pairs))
                for a, (i, j) in zip(accs, pairs):
                    plsc.store_scatter(out_v, [bo + _PAIR_IDX[(i, j)]], a)

        def phase(c, cidx, x_v, out_v, sa, sb, out_s):
            wait_in(x_v, sa, sb)

            @pl.when(c > 0)
            def _():
                pltpu.make_async_copy(out_v, out_hbm.at[pl.ds(0, OW)],
                                      out_s).wait()

            compute(x_v, out_v)
            pltpu.async_copy(out_v, out_hbm.at[pl.ds((base + cidx * L) * P,
                                                     OW)], out_s)
            nxt = cidx + 2

            @pl.when(nxt < nchunks)
            def _():
                issue_in(nxt, x_v, sa, sb)

        issue_in(0, x_v0, in_sa0, in_sb0)
        issue_in(1, x_v1, in_sa1, in_sb1)

        def step(c, carry):
            phase(c, 2 * c, x_v0, out_v0, in_sa0, in_sb0, out_s0)
            phase(c, 2 * c + 1, x_v1, out_v1, in_sa1, in_sb1, out_s1)
            return carry

        lax.fori_loop(0, nphase, step, 0)
        pltpu.make_async_copy(out_v0, out_hbm.at[pl.ds(0, OW)], out_s0).wait()
        pltpu.make_async_copy(out_v1, out_hbm.at[pl.ds(0, OW)], out_s1).wait()

    return body


def kernel(inputs):
    b_total = inputs.shape[0]
    mesh = plsc.VectorSubcoreMesh(core_axis_name="c", subcore_axis_name="s")
    kfn = pl.kernel(
        _body(b_total),
        mesh=mesh,
        out_type=jax.ShapeDtypeStruct((b_total * P,), jnp.float32),
        scratch_types=[
            pltpu.VMEM((XW,), jnp.float32),
            pltpu.VMEM((XW,), jnp.float32),
            pltpu.VMEM((OW,), jnp.float32),
            pltpu.VMEM((OW,), jnp.float32),
            pltpu.SemaphoreType.DMA,
            pltpu.SemaphoreType.DMA,
            pltpu.SemaphoreType.DMA,
            pltpu.SemaphoreType.DMA,
            pltpu.SemaphoreType.DMA,
            pltpu.SemaphoreType.DMA,
        ],
        compiler_params=pltpu.CompilerParams(needs_layout_passes=False),
    )
    return kfn(inputs.reshape(b_total * F * D)).reshape(b_total, P)


# speedup vs baseline: 1.4768x; 1.4768x over previous
import jax
import jax.numpy as jnp
from jax import lax
from jax.experimental import pallas as pl
from jax.experimental.pallas import tpu as pltpu
from jax.experimental.pallas import tpu_sc as plsc

F = 26
D = 64
P = F * (F - 1) // 2
L = 16
NC = 2
NS = 16
NW = NC * NS
XW = L * F * D
OW = L * P
H = XW // 2

_PAIR_IDX = {}
for _i in range(F - 1):
    for _j in range(_i + 1, F):
        _PAIR_IDX[(_i, _j)] = len(_PAIR_IDX)

_BLOCKS = [(0, 6), (6, 12), (12, 18), (18, 24), (24, 26)]

_SCHED = []
for _bi in range(len(_BLOCKS)):
    _fi = list(range(*_BLOCKS[_bi]))
    _diag = [(i, j) for i in _fi for j in _fi if i < j]
    if _diag:
        _SCHED.append((_fi, _diag))
    for _bj in range(_bi + 1, len(_BLOCKS)):
        _fj = list(range(*_BLOCKS[_bj]))
        _SCHED.append((_fi + _fj, [(i, j) for i in _fi for j in _fj]))

assert sorted(p for _, ps in _SCHED for p in ps) == sorted(_PAIR_IDX)


def _body(b_total):
    rows_per_w = b_total // NW
    nchunks = rows_per_w // L
    nphase = nchunks // 2

    def body(x_hbm, out_hbm, x_v0, x_v1, out_v0, out_v1,
             in_sa0, in_sb0, in_sa1, in_sb1, out_s0, out_s1):
        wid = lax.axis_index("s") * NC + lax.axis_index("c")
        base = wid * rows_per_w
        b_iota = lax.iota(jnp.int32, L)
        bx = b_iota * (F * D)
        bo = b_iota * P

        def issue_in(cidx, x_v, sa, sb):
            off = (base + cidx * L) * (F * D)
            pltpu.async_copy(x_hbm.at[pl.ds(off, H)], x_v.at[pl.ds(0, H)], sa)
            pltpu.async_copy(x_hbm.at[pl.ds(off + H, H)],
                             x_v.at[pl.ds(H, H)], sb)

        def wait_in(x_v, sa, sb):
            pltpu.make_async_copy(x_hbm.at[pl.ds(0, H)],
                                  x_v.at[pl.ds(0, H)], sa).wait()
            pltpu.make_async_copy(x_hbm.at[pl.ds(0, H)],
                                  x_v.at[pl.ds(H, H)], sb).wait()

        def compute(x_v, out_v):
            for fields, pairs in _SCHED:

                def dstep(d, accs, fields=fields, pairs=pairs):
                    rot = (jnp.full((L,), d, jnp.int32) + b_iota) & (D - 1)
                    vidx = bx + rot
                    vals = {
                        f: plsc.load_gather(
                            x_v.at[pl.ds(f * D, XW - f * D)], [vidx])
                        for f in fields
                    }
                    return tuple(a + vals[i] * vals[j]
                                 for a, (i, j) in zip(accs, pairs))

                accs = lax.fori_loop(
                    0, D, dstep,
                    tuple(jnp.zeros((L,), jnp.float32) for _ in pairs))
                for a, (i, j) in zip(accs, pairs):
                    plsc.store_scatter(out_v, [bo + _PAIR_IDX[(i, j)]], a)

        def phase(c, cidx, x_v, out_v, sa, sb, out_s):
            wait_in(x_v, sa, sb)

            @pl.when(c > 0)
            def _():
                pltpu.make_async_copy(out_v, out_hbm.at[pl.ds(0, OW)],
                                      out_s).wait()

            compute(x_v, out_v)
            pltpu.async_copy(out_v, out_hbm.at[pl.ds((base + cidx * L) * P,
                                                     OW)], out_s)
            nxt = cidx + 2

            @pl.when(nxt < nchunks)
            def _():
                issue_in(nxt, x_v, sa, sb)

        issue_in(0, x_v0, in_sa0, in_sb0)
        issue_in(1, x_v1, in_sa1, in_sb1)

        def step(c, carry):
            phase(c, 2 * c, x_v0, out_v0, in_sa0, in_sb0, out_s0)
            phase(c, 2 * c + 1, x_v1, out_v1, in_sa1, in_sb1, out_s1)
            return carry

        lax.fori_loop(0, nphase, step, 0)
        pltpu.make_async_copy(out_v0, out_hbm.at[pl.ds(0, OW)], out_s0).wait()
        pltpu.make_async_copy(out_v1, out_hbm.at[pl.ds(0, OW)], out_s1).wait()

    return body


def kernel(inputs):
    b_total = inputs.shape[0]
    mesh = plsc.VectorSubcoreMesh(core_axis_name="c", subcore_axis_name="s")
    kfn = pl.kernel(
        _body(b_total),
        mesh=mesh,
        out_type=jax.ShapeDtypeStruct((b_total * P,), jnp.float32),
        scratch_types=[
            pltpu.VMEM((XW,), jnp.float32),
            pltpu.VMEM((XW,), jnp.float32),
            pltpu.VMEM((OW,), jnp.float32),
            pltpu.VMEM((OW,), jnp.float32),
            pltpu.SemaphoreType.DMA,
            pltpu.SemaphoreType.DMA,
            pltpu.SemaphoreType.DMA,
            pltpu.SemaphoreType.DMA,
            pltpu.SemaphoreType.DMA,
            pltpu.SemaphoreType.DMA,
        ],
        compiler_params=pltpu.CompilerParams(needs_layout_passes=False),
    )
    return kfn(inputs.reshape(b_total * F * D)).reshape(b_total, P)
